# baseline (speedup 1.0000x reference)
import functools

import jax
import jax.numpy as jnp
import numpy as np
from jax.experimental import pallas as pl

NJ = 128
NM = 32
TOT = NJ + NM
FPAD = 8
ROWS_PER_STEP = 4
KDIM = NM * NM


def _prelude_kernel(h_col_ref, h_row_ref, hl8_ref, wpn_ref, g_ref, gt_ref,
                    wcj_ref, bcj_ref, wcm_ref, bcm_ref, ep_ref, eq_ref,
                    wljt_ref, blj_ref, bm2_ref, v_out_ref, base_out_ref):
    f32 = jnp.float32
    h_col = h_col_ref[...]
    h_row = h_row_ref[...]

    i32 = jnp.int32
    j_col = jax.lax.broadcasted_iota(i32, (NJ, NJ), 0)
    i_row = jax.lax.broadcasted_iota(i32, (NJ, NJ), 1)
    gt = (h_col > h_row).astype(f32)
    eq = (h_col == h_row).astype(f32)
    cmp = gt + eq * (j_col < i_row).astype(f32)
    rank_row = jnp.sum(cmp, axis=0, keepdims=True).astype(i32)
    perm = (j_col == rank_row).astype(f32)

    sorted8 = jnp.dot(perm, hl8_ref[...], preferred_element_type=f32)
    feats = sorted8 + wpn_ref[...]

    a = (g_ref[...] != 0.0).astype(f32)
    at = (gt_ref[...] != 0.0).astype(f32)
    ajj_t = at[:NJ, :]
    ajm_t = at[NJ:, :]

    od_jj = jnp.sum(a[:, :NJ], axis=1, keepdims=True)
    od_jm = jnp.sum(a[:, NJ:], axis=1, keepdims=True)
    id_jj = jnp.sum(ajj_t, axis=1, keepdims=True)
    id_jm = jnp.sum(ajm_t, axis=1, keepdims=True)

    pad_jj = float(NJ * NJ) - jnp.sum(id_jj, axis=0, keepdims=True)
    pad_jm = float(NJ * NM) - jnp.sum(id_jm, axis=0, keepdims=True)
    one0_128 = (jax.lax.broadcasted_iota(i32, (NJ, 1), 0) == 0).astype(f32)
    one0_32 = (jax.lax.broadcasted_iota(i32, (NM, 1), 0) == 0).astype(f32)

    od_jj = jnp.maximum(od_jj + pad_jj * one0_128, 1.0)
    id_jj = jnp.maximum(id_jj + pad_jj * one0_128, 1.0)
    od_jm = jnp.maximum(od_jm + pad_jm * one0_128, 1.0)
    id_jm = jnp.maximum(id_jm + pad_jm * one0_32, 1.0)

    m_jj = feats * jax.lax.rsqrt(od_jj)
    agg_jj = jnp.dot(ajj_t, m_jj, preferred_element_type=f32)
    agg_jj = agg_jj + pad_jj * (one0_128 * m_jj[0:1, :])
    jc = jnp.dot(agg_jj * jax.lax.rsqrt(id_jj), wcj_ref[...],
                 preferred_element_type=f32) + bcj_ref[...]

    m_jm = feats * jax.lax.rsqrt(od_jm)
    agg_jm = jnp.dot(ajm_t, m_jm, preferred_element_type=f32)
    agg_jm = agg_jm + pad_jm * (one0_32 * m_jm[0:1, :])
    mach = jnp.dot(agg_jm * jax.lax.rsqrt(id_jm), wcm_ref[...],
                   preferred_element_type=f32) + bcm_ref[...]

    v_out_ref[...] = jnp.sum(
        ep_ref[...] * jnp.dot(mach, eq_ref[...], preferred_element_type=f32),
        axis=0, keepdims=True)

    base_out_ref[...] = (jnp.dot(jc, wljt_ref[...], preferred_element_type=f32)
                         + blj_ref[...] + bm2_ref[...])


NSTREAMS = 2


def _gemv_kernel(*refs):
    m_refs = refs[:NSTREAMS]
    v_ref, base_ref, out_ref = refs[NSTREAMS:]
    v = v_ref[...].reshape(1, 1, KDIM)
    for s in range(NSTREAMS):
        lo = s * ROWS_PER_STEP
        hi = lo + ROWS_PER_STEP
        out_ref[lo:hi, :] = (jnp.sum(m_refs[s][...] * v, axis=2)
                             + base_ref[lo:hi, :])


@jax.jit
def kernel(h, L, W, P, N, Graph, W_conv_job, b_conv_job, W_conv_mach,
           b_conv_mach, W_lin_job, b_lin_job, W_lin_mach, b_lin_mach):
    f32 = jnp.float32
    h_col = h.reshape(NJ, 1)
    h_row = h.reshape(1, NJ)
    hl8 = jnp.concatenate(
        [h.reshape(NJ, 1), L.reshape(NJ, 1), jnp.zeros((NJ, FPAD - 2), f32)],
        axis=1)
    wpn = jnp.concatenate(
        [jnp.zeros((2,), f32), jnp.stack([W, P, N]).astype(f32),
         jnp.zeros((FPAD - 5,), f32)]).reshape(1, FPAD)
    wcj8 = jnp.concatenate([W_conv_job, jnp.zeros((FPAD - 5, NJ), f32)], axis=0)
    wcm8 = jnp.concatenate([W_conv_mach, jnp.zeros((FPAD - 5, NM), f32)], axis=0)

    jidx = np.arange(KDIM)
    ep = jnp.asarray((jidx // NM)[None, :] == np.arange(NM)[:, None],
                     dtype=f32)
    eq = jnp.asarray((jidx % NM)[None, :] == np.arange(NM)[:, None],
                     dtype=f32)

    v_row, base = pl.pallas_call(
        _prelude_kernel,
        out_shape=(
            jax.ShapeDtypeStruct((1, KDIM), f32),
            jax.ShapeDtypeStruct((NJ, TOT), f32),
        ),
    )(h_col, h_row, hl8, wpn, Graph, Graph.T, wcj8, b_conv_job.reshape(1, NJ),
      wcm8, b_conv_mach.reshape(1, NM), ep, eq, W_lin_job.T,
      b_lin_job.reshape(1, TOT), b_lin_mach.reshape(NJ, TOT))

    m3 = W_lin_mach.reshape(NJ, TOT, KDIM)
    rows_per_iter = NSTREAMS * ROWS_PER_STEP
    grid = NJ // rows_per_iter

    def _mk_map(s):
        return lambda i: (NSTREAMS * i + s, 0, 0)

    out = pl.pallas_call(
        _gemv_kernel,
        grid=(grid,),
        in_specs=[pl.BlockSpec((ROWS_PER_STEP, TOT, KDIM), _mk_map(s))
                  for s in range(NSTREAMS)]
        + [
            pl.BlockSpec((1, KDIM), lambda i: (0, 0)),
            pl.BlockSpec((rows_per_iter, TOT), lambda i: (i, 0)),
        ],
        out_specs=pl.BlockSpec((rows_per_iter, TOT), lambda i: (i, 0)),
        out_shape=jax.ShapeDtypeStruct((NJ, TOT), f32),
    )(*([m3] * NSTREAMS), v_row, base)
    return out


# speedup vs baseline: 1.0600x; 1.0600x over previous
import functools

import jax
import jax.numpy as jnp
import numpy as np
from jax.experimental import pallas as pl

NJ = 128
NM = 32
TOT = NJ + NM
FPAD = 8
ROWS_PER_STEP = 8
KDIM = NM * NM


def _prelude_compute(h_col_ref, h_row_ref, hl8_ref, wpn_ref, g_ref, gt_ref,
                    wcj_ref, bcj_ref, wcm_ref, bcm_ref, ep_ref, eq_ref,
                    wljt_ref, blj_ref, bm2_ref, v_out_ref, base_out_ref):
    f32 = jnp.float32
    h_col = h_col_ref[...]
    h_row = h_row_ref[...]

    i32 = jnp.int32
    j_col = jax.lax.broadcasted_iota(i32, (NJ, NJ), 0)
    i_row = jax.lax.broadcasted_iota(i32, (NJ, NJ), 1)
    gt = (h_col > h_row).astype(f32)
    eq = (h_col == h_row).astype(f32)
    cmp = gt + eq * (j_col < i_row).astype(f32)
    rank_row = jnp.sum(cmp, axis=0, keepdims=True).astype(i32)
    perm = (j_col == rank_row).astype(f32)

    sorted8 = jnp.dot(perm, hl8_ref[...], preferred_element_type=f32)
    feats = sorted8 + wpn_ref[...]

    a = (g_ref[...] != 0.0).astype(f32)
    at = (gt_ref[...] != 0.0).astype(f32)
    ajj_t = at[:NJ, :]
    ajm_t = at[NJ:, :]

    od_jj = jnp.sum(a[:, :NJ], axis=1, keepdims=True)
    od_jm = jnp.sum(a[:, NJ:], axis=1, keepdims=True)
    id_jj = jnp.sum(ajj_t, axis=1, keepdims=True)
    id_jm = jnp.sum(ajm_t, axis=1, keepdims=True)

    pad_jj = float(NJ * NJ) - jnp.sum(id_jj, axis=0, keepdims=True)
    pad_jm = float(NJ * NM) - jnp.sum(id_jm, axis=0, keepdims=True)
    one0_128 = (jax.lax.broadcasted_iota(i32, (NJ, 1), 0) == 0).astype(f32)
    one0_32 = (jax.lax.broadcasted_iota(i32, (NM, 1), 0) == 0).astype(f32)

    od_jj = jnp.maximum(od_jj + pad_jj * one0_128, 1.0)
    id_jj = jnp.maximum(id_jj + pad_jj * one0_128, 1.0)
    od_jm = jnp.maximum(od_jm + pad_jm * one0_128, 1.0)
    id_jm = jnp.maximum(id_jm + pad_jm * one0_32, 1.0)

    m_jj = feats * jax.lax.rsqrt(od_jj)
    agg_jj = jnp.dot(ajj_t, m_jj, preferred_element_type=f32)
    agg_jj = agg_jj + pad_jj * (one0_128 * m_jj[0:1, :])
    jc = jnp.dot(agg_jj * jax.lax.rsqrt(id_jj), wcj_ref[...],
                 preferred_element_type=f32) + bcj_ref[...]

    m_jm = feats * jax.lax.rsqrt(od_jm)
    agg_jm = jnp.dot(ajm_t, m_jm, preferred_element_type=f32)
    agg_jm = agg_jm + pad_jm * (one0_32 * m_jm[0:1, :])
    mach = jnp.dot(agg_jm * jax.lax.rsqrt(id_jm), wcm_ref[...],
                   preferred_element_type=f32) + bcm_ref[...]

    v_out_ref[...] = jnp.sum(
        ep_ref[...] * jnp.dot(mach, eq_ref[...], preferred_element_type=f32),
        axis=0, keepdims=True)

    base_out_ref[...] = (jnp.dot(jc, wljt_ref[...], preferred_element_type=f32)
                         + blj_ref[...] + bm2_ref[...])


NSTREAMS = 2


def _fused_kernel(*refs):
    (h_col_ref, h_row_ref, hl8_ref, wpn_ref, g_ref, gt_ref, wcj_ref, bcj_ref,
     wcm_ref, bcm_ref, ep_ref, eq_ref, wljt_ref, blj_ref, bm2_ref) = refs[:15]
    m_refs = refs[15:15 + NSTREAMS]
    out_ref, v_s, base_s = refs[15 + NSTREAMS:]
    i = pl.program_id(0)

    @pl.when(i == 0)
    def _prelude():
        _prelude_compute(h_col_ref, h_row_ref, hl8_ref, wpn_ref, g_ref,
                         gt_ref, wcj_ref, bcj_ref, wcm_ref, bcm_ref, ep_ref,
                         eq_ref, wljt_ref, blj_ref, bm2_ref, v_s, base_s)

    v = v_s[...].reshape(1, 1, KDIM)
    rows_per_iter = NSTREAMS * ROWS_PER_STEP
    for s in range(NSTREAMS):
        lo = s * ROWS_PER_STEP
        hi = lo + ROWS_PER_STEP
        out_ref[lo:hi, :] = (
            jnp.sum(m_refs[s][...] * v, axis=2)
            + base_s[pl.ds(i * rows_per_iter + lo, ROWS_PER_STEP), :])


@jax.jit
def kernel(h, L, W, P, N, Graph, W_conv_job, b_conv_job, W_conv_mach,
           b_conv_mach, W_lin_job, b_lin_job, W_lin_mach, b_lin_mach):
    f32 = jnp.float32
    h_col = h.reshape(NJ, 1)
    h_row = h.reshape(1, NJ)
    hl8 = jnp.concatenate(
        [h.reshape(NJ, 1), L.reshape(NJ, 1), jnp.zeros((NJ, FPAD - 2), f32)],
        axis=1)
    wpn = jnp.concatenate(
        [jnp.zeros((2,), f32), jnp.stack([W, P, N]).astype(f32),
         jnp.zeros((FPAD - 5,), f32)]).reshape(1, FPAD)
    wcj8 = jnp.concatenate([W_conv_job, jnp.zeros((FPAD - 5, NJ), f32)], axis=0)
    wcm8 = jnp.concatenate([W_conv_mach, jnp.zeros((FPAD - 5, NM), f32)], axis=0)

    jidx = np.arange(KDIM)
    ep = jnp.asarray((jidx // NM)[None, :] == np.arange(NM)[:, None],
                     dtype=f32)
    eq = jnp.asarray((jidx % NM)[None, :] == np.arange(NM)[:, None],
                     dtype=f32)

    m3 = W_lin_mach.reshape(NJ, TOT, KDIM)
    rows_per_iter = NSTREAMS * ROWS_PER_STEP
    grid = NJ // rows_per_iter

    def _mk_map(s):
        return lambda i: (NSTREAMS * i + s, 0, 0)

    def _whole(shape):
        nd = len(shape)
        return pl.BlockSpec(shape, lambda i, _n=nd: (0,) * _n)

    prelude_args = (h_col, h_row, hl8, wpn, Graph, Graph.T, wcj8,
                    b_conv_job.reshape(1, NJ), wcm8, b_conv_mach.reshape(1, NM),
                    ep, eq, W_lin_job.T, b_lin_job.reshape(1, TOT),
                    b_lin_mach.reshape(NJ, TOT))

    from jax.experimental.pallas import tpu as pltpu
    out = pl.pallas_call(
        _fused_kernel,
        grid=(grid,),
        in_specs=[_whole(a.shape) for a in prelude_args]
        + [pl.BlockSpec((ROWS_PER_STEP, TOT, KDIM), _mk_map(s))
           for s in range(NSTREAMS)],
        out_specs=pl.BlockSpec((rows_per_iter, TOT), lambda i: (i, 0)),
        out_shape=jax.ShapeDtypeStruct((NJ, TOT), f32),
        scratch_shapes=[
            pltpu.VMEM((1, KDIM), f32),
            pltpu.VMEM((NJ, TOT), f32),
        ],
    )(*prelude_args, *([m3] * NSTREAMS))
    return out
